# Initial kernel scaffold; baseline (speedup 1.0000x reference)
import jax
import jax.numpy as jnp
from jax.experimental import pallas as pl


def kernel(x, edge_index, W1, b1, W2, b2):
    raise NotImplementedError("write your pallas kernel here")


# speedup vs baseline: 11.6783x; 11.6783x over previous
import functools

import jax
import jax.numpy as jnp
from jax import lax
from jax.experimental import pallas as pl
from jax.experimental.pallas import tpu as pltpu
from jax.experimental.pallas import tpu_sc as plsc

N_NODES = 10000
NPAD = 10240
NC = 2
NS = 16
NW = NC * NS
CH = 128
ROWS_PER_TILE = NPAD // NS
DEG_W = 16
BM = 1024


def _make_sc_agg(D, K, gather):
    mesh = plsc.VectorSubcoreMesh(core_axis_name="c", subcore_axis_name="s")

    scratch = [
        pltpu.VMEM((CH,), jnp.int32),
        pltpu.VMEM((CH,), jnp.int32),
        pltpu.VMEM((CH, D), jnp.float32),
        pltpu.VMEM_SHARED((NPAD, D), jnp.float32),
        pltpu.SemaphoreType.DMA,
    ]

    @functools.partial(
        pl.kernel,
        mesh=mesh,
        out_type=jax.ShapeDtypeStruct((NC, NPAD, D), jnp.float32),
        scratch_types=scratch,
        compiler_params=pltpu.CompilerParams(use_tc_tiling_on_sc=False),
    )
    def agg(tab_hbm, src_hbm, dst_hbm, zeros_hbm, out_hbm,
            sidx_v, didx_v, rows_v, acc, sem):
        c = lax.axis_index("c")
        s = lax.axis_index("s")
        wid = c * NS + s
        r0 = s * ROWS_PER_TILE
        pltpu.sync_copy(zeros_hbm.at[pl.ds(r0, ROWS_PER_TILE)],
                        acc.at[pl.ds(r0, ROWS_PER_TILE)])
        if not gather:
            ones = jnp.full((16,), 1.0, jnp.float32)
            for i in range(CH):
                rows_v[i, :] = ones
        plsc.subcore_barrier()

        def body(j, carry):
            if gather:
                pltpu.sync_copy(src_hbm.at[wid, j], sidx_v)
                pltpu.async_copy(tab_hbm.at[sidx_v], rows_v, sem).wait()
            pltpu.sync_copy(dst_hbm.at[wid, j], didx_v)
            pltpu.sync_copy(rows_v, acc.at[didx_v], add=True)
            return carry

        lax.fori_loop(0, K, body, 0)
        plsc.subcore_barrier()
        pltpu.sync_copy(acc.at[pl.ds(r0, ROWS_PER_TILE)],
                        out_hbm.at[c, pl.ds(r0, ROWS_PER_TILE)])

    return agg


def _tc_matmul(x, w):
    m, kdim = x.shape
    n = w.shape[1]

    def body(x_ref, w_ref, o_ref):
        o_ref[...] = jnp.dot(x_ref[...], w_ref[...],
                             preferred_element_type=jnp.float32)

    return pl.pallas_call(
        body,
        grid=(m // BM,),
        in_specs=[
            pl.BlockSpec((BM, kdim), lambda i: (i, 0)),
            pl.BlockSpec((kdim, n), lambda i: (0, 0)),
        ],
        out_specs=pl.BlockSpec((BM, n), lambda i: (i, 0)),
        out_shape=jax.ShapeDtypeStruct((m, n), jnp.float32),
    )(x, w)


def _tc_scale(h, d0, d1):
    m, n = h.shape

    def body(h_ref, d0_ref, d1_ref, o_ref):
        dinv = lax.rsqrt(d0_ref[...] + d1_ref[...] + 1.0)
        o_ref[...] = h_ref[...] * dinv

    return pl.pallas_call(
        body,
        grid=(m // BM,),
        in_specs=[
            pl.BlockSpec((BM, n), lambda i: (i, 0)),
            pl.BlockSpec((BM, 1), lambda i: (i, 0)),
            pl.BlockSpec((BM, 1), lambda i: (i, 0)),
        ],
        out_specs=pl.BlockSpec((BM, n), lambda i: (i, 0)),
        out_shape=jax.ShapeDtypeStruct((m, n), jnp.float32),
    )(h, d0, d1)


def _tc_mid(p0, p1, hs1, d0, d1, b1, w2):
    m, n = hs1.shape
    n2 = w2.shape[1]

    def body(p0_ref, p1_ref, hs1_ref, d0_ref, d1_ref, b1_ref, w2_ref, o_ref):
        dinv = lax.rsqrt(d0_ref[...] + d1_ref[...] + 1.0)
        out1 = dinv * (p0_ref[...] + p1_ref[...] + hs1_ref[...]) + b1_ref[...]
        a = jnp.maximum(out1, 0.0)
        o_ref[...] = dinv * jnp.dot(a, w2_ref[...],
                                    preferred_element_type=jnp.float32)

    return pl.pallas_call(
        body,
        grid=(m // BM,),
        in_specs=[
            pl.BlockSpec((BM, n), lambda i: (i, 0)),
            pl.BlockSpec((BM, n), lambda i: (i, 0)),
            pl.BlockSpec((BM, n), lambda i: (i, 0)),
            pl.BlockSpec((BM, 1), lambda i: (i, 0)),
            pl.BlockSpec((BM, 1), lambda i: (i, 0)),
            pl.BlockSpec((1, n), lambda i: (0, 0)),
            pl.BlockSpec((n, n2), lambda i: (0, 0)),
        ],
        out_specs=pl.BlockSpec((BM, n2), lambda i: (i, 0)),
        out_shape=jax.ShapeDtypeStruct((m, n2), jnp.float32),
    )(p0, p1, hs1, d0, d1, b1, w2)


def _tc_final(p0, p1, hs2, d0, d1, b2):
    m, n = hs2.shape

    def body(p0_ref, p1_ref, hs2_ref, d0_ref, d1_ref, b2_ref, o_ref):
        dinv = lax.rsqrt(d0_ref[...] + d1_ref[...] + 1.0)
        o_ref[...] = dinv * (p0_ref[...] + p1_ref[...] + hs2_ref[...]) + b2_ref[...]

    return pl.pallas_call(
        body,
        grid=(m // BM,),
        in_specs=[
            pl.BlockSpec((BM, n), lambda i: (i, 0)),
            pl.BlockSpec((BM, n), lambda i: (i, 0)),
            pl.BlockSpec((BM, n), lambda i: (i, 0)),
            pl.BlockSpec((BM, 1), lambda i: (i, 0)),
            pl.BlockSpec((BM, 1), lambda i: (i, 0)),
            pl.BlockSpec((1, n), lambda i: (0, 0)),
        ],
        out_specs=pl.BlockSpec((BM, n), lambda i: (i, 0)),
        out_shape=jax.ShapeDtypeStruct((m, n), jnp.float32),
    )(p0, p1, hs2, d0, d1, b2)


def kernel(x, edge_index, W1, b1, W2, b2):
    n, in_dim = x.shape
    hid = W1.shape[1]
    out_dim = W2.shape[1]
    e = edge_index.shape[1]
    k = -(-e // (NW * CH))
    epad = NW * CH * k

    src = edge_index[0].astype(jnp.int32)
    dst = edge_index[1].astype(jnp.int32)
    pad = epad - e
    fill = jnp.full((pad,), n, jnp.int32)
    src_p = jnp.concatenate([src, fill]).reshape(NW, k, CH)
    dst_p = jnp.concatenate([dst, fill]).reshape(NW, k, CH)

    x_p = jnp.pad(x, ((0, NPAD - n), (0, 0)))
    b1r = b1.reshape(1, hid)
    b2r = b2.reshape(1, out_dim)

    ones_tab = jnp.ones((NPAD, DEG_W), jnp.float32)
    zeros_w = jnp.zeros((NPAD, DEG_W), jnp.float32)
    zeros_h = jnp.zeros((NPAD, hid), jnp.float32)
    zeros_o = jnp.zeros((NPAD, out_dim), jnp.float32)

    deg_fn = _make_sc_agg(DEG_W, k, gather=False)
    pdeg = deg_fn(ones_tab, src_p, dst_p, zeros_w)
    h1 = _tc_matmul(x_p, W1)

    d0 = pdeg[0, :, 0:1]
    d1 = pdeg[1, :, 0:1]

    hs1 = _tc_scale(h1, d0, d1)
    agg1_fn = _make_sc_agg(hid, k, gather=True)
    p1 = agg1_fn(hs1, src_p, dst_p, zeros_h)

    hs2 = _tc_mid(p1[0], p1[1], hs1, d0, d1, b1r, W2)
    agg2_fn = _make_sc_agg(out_dim, k, gather=True)
    p2 = agg2_fn(hs2, src_p, dst_p, zeros_o)

    z = _tc_final(p2[0], p2[1], hs2, d0, d1, b2r)
    return z[:n]
